# Initial kernel scaffold; baseline (speedup 1.0000x reference)
import functools

import jax
import jax.numpy as jnp
from jax import lax
from jax.experimental import pallas as pl
from jax.experimental.pallas import tpu as pltpu
from jax.experimental.pallas import tpu_sc as plsc

NUM_CORES = 2
NUM_SUBCORES = 16
NUM_WORKERS = NUM_CORES * NUM_SUBCORES
CHUNK = 128
K = 16


@functools.lru_cache(maxsize=None)
def _build(num_chunks, vocab, dim):
    chunks_per_w = num_chunks // NUM_WORKERS
    n_steps = chunks_per_w // K
    mesh = plsc.VectorSubcoreMesh(
        core_axis_name="c", subcore_axis_name="s")

    @functools.partial(
        pl.kernel,
        out_type=jax.ShapeDtypeStruct((num_chunks, CHUNK, dim), jnp.float32),
        mesh=mesh,
        scratch_types=[
            pltpu.VMEM((K, CHUNK), jnp.int32),
            pltpu.VMEM((K, CHUNK, dim), jnp.float32),
            pltpu.SemaphoreType.DMA,
        ],
    )
    def emb(idx_hbm, table_hbm, out_hbm, idx_v, rows_v, sem):
        wid = lax.axis_index("s") * NUM_CORES + lax.axis_index("c")
        row0 = wid * chunks_per_w

        def step(g, carry):
            r = row0 + g * K
            pltpu.sync_copy(idx_hbm.at[pl.ds(r, K)], idx_v)
            copies = [
                pltpu.async_copy(table_hbm.at[idx_v.at[j]], rows_v.at[j], sem)
                for j in range(K)
            ]
            for c in copies:
                c.wait()
            pltpu.sync_copy(rows_v, out_hbm.at[pl.ds(r, K)])
            return carry

        lax.fori_loop(0, n_steps, step, 0)

    return emb


def kernel(token_ids, weight):
    shape = token_ids.shape
    b = token_ids.size
    num_chunks = b // CHUNK
    dim = weight.shape[1]
    idx = token_ids.reshape(num_chunks, CHUNK)
    out = _build(num_chunks, weight.shape[0], dim)(idx, weight)
    return out.reshape(*shape, dim)


# speedup vs baseline: 4.9469x; 4.9469x over previous
import functools

import jax
import jax.numpy as jnp
from jax import lax
from jax.experimental import pallas as pl
from jax.experimental.pallas import tpu as pltpu
from jax.experimental.pallas import tpu_sc as plsc

NUM_CORES = 2
NUM_SUBCORES = 16
NUM_WORKERS = NUM_CORES * NUM_SUBCORES
CHUNK = 128
K = 16


@functools.lru_cache(maxsize=None)
def _build(num_chunks, vocab, dim):
    chunks_per_w = num_chunks // NUM_WORKERS
    n_steps = chunks_per_w // K
    mesh = plsc.VectorSubcoreMesh(
        core_axis_name="c", subcore_axis_name="s")

    @functools.partial(
        pl.kernel,
        out_type=jax.ShapeDtypeStruct((num_chunks, CHUNK, dim), jnp.float32),
        mesh=mesh,
        scratch_types=[
            pltpu.VMEM((K, CHUNK), jnp.int32),
            pltpu.VMEM((K, CHUNK, dim), jnp.float32),
            pltpu.SemaphoreType.DMA,
        ],
        compiler_params=pltpu.CompilerParams(use_tc_tiling_on_sc=False),
    )
    def emb(idx_hbm, table_hbm, out_hbm, idx_v, rows_v, sem):
        wid = lax.axis_index("s") * NUM_CORES + lax.axis_index("c")
        row0 = wid * chunks_per_w

        def step(g, carry):
            r = row0 + g * K
            pltpu.sync_copy(idx_hbm.at[pl.ds(r, K)], idx_v)
            copies = [
                pltpu.async_copy(table_hbm.at[idx_v.at[j]], rows_v.at[j], sem)
                for j in range(K)
            ]
            for c in copies:
                c.wait()
            pltpu.sync_copy(rows_v, out_hbm.at[pl.ds(r, K)])
            return carry

        lax.fori_loop(0, n_steps, step, 0)

    return emb


def kernel(token_ids, weight):
    shape = token_ids.shape
    b = token_ids.size
    num_chunks = b // CHUNK
    dim = weight.shape[1]
    idx = token_ids.reshape(num_chunks, CHUNK)
    out = _build(num_chunks, weight.shape[0], dim)(idx, weight)
    return out.reshape(*shape, dim)


# speedup vs baseline: 4.9712x; 1.0049x over previous
import functools

import jax
import jax.numpy as jnp
from jax import lax
from jax.experimental import pallas as pl
from jax.experimental.pallas import tpu as pltpu
from jax.experimental.pallas import tpu_sc as plsc

NUM_CORES = 2
NUM_SUBCORES = 16
NUM_WORKERS = NUM_CORES * NUM_SUBCORES
CHUNK = 128
K = 10
NBUF = 2


@functools.lru_cache(maxsize=None)
def _build(num_chunks, vocab, dim):
    chunks_per_w = num_chunks // NUM_WORKERS
    n_steps = chunks_per_w // K
    n_outer = n_steps // NBUF
    mesh = plsc.VectorSubcoreMesh(
        core_axis_name="c", subcore_axis_name="s")

    @functools.partial(
        pl.kernel,
        out_type=jax.ShapeDtypeStruct((num_chunks, CHUNK, dim), jnp.float32),
        mesh=mesh,
        scratch_types=[
            pltpu.VMEM((NBUF, K, CHUNK), jnp.int32),
            pltpu.VMEM((NBUF, K, CHUNK, dim), jnp.float32),
            pltpu.SemaphoreType.DMA,
            pltpu.SemaphoreType.DMA,
            pltpu.SemaphoreType.DMA,
            pltpu.SemaphoreType.DMA,
        ],
        compiler_params=pltpu.CompilerParams(use_tc_tiling_on_sc=False),
    )
    def emb(idx_hbm, table_hbm, out_hbm, idx_v, rows_v, g0, g1, o0, o1):
        wid = lax.axis_index("s") * NUM_CORES + lax.axis_index("c")
        row0 = wid * chunks_per_w
        gsems = (g0, g1)
        osems = (o0, o1)

        def fire(blk, s):
            r = row0 + blk * K
            pltpu.sync_copy(idx_hbm.at[pl.ds(r, K)], idx_v.at[s])
            for j in range(K):
                pltpu.async_copy(
                    table_hbm.at[idx_v.at[s].at[j]], rows_v.at[s].at[j],
                    gsems[s])

        def drain_gathers(s):
            for j in range(K):
                pltpu.make_async_copy(
                    table_hbm.at[idx_v.at[s].at[j]], rows_v.at[s].at[j],
                    gsems[s]).wait()

        def fire_store(blk, s):
            r = row0 + blk * K
            pltpu.async_copy(rows_v.at[s], out_hbm.at[pl.ds(r, K)], osems[s])

        def wait_store(s):
            pltpu.make_async_copy(
                rows_v.at[s], out_hbm.at[pl.ds(row0, K)], osems[s]).wait()

        fire(0, 0)

        def outer(t, carry):
            for b in range(NBUF):
                s, o = b, 1 - b
                blk = t * NBUF + b

                @pl.when(blk + 1 < n_steps)
                def _fire_next():
                    @pl.when(blk >= 1)
                    def _wait_prev_store():
                        wait_store(o)
                    fire(blk + 1, o)

                drain_gathers(s)
                fire_store(blk, s)
            return carry

        lax.fori_loop(0, n_outer, outer, 0)
        wait_store(0)
        wait_store(1)

    return emb


def kernel(token_ids, weight):
    shape = token_ids.shape
    b = token_ids.size
    num_chunks = b // CHUNK
    dim = weight.shape[1]
    idx = token_ids.reshape(num_chunks, CHUNK)
    out = _build(num_chunks, weight.shape[0], dim)(idx, weight)
    return out.reshape(*shape, dim)


# speedup vs baseline: 6.6800x; 1.3437x over previous
import functools

import jax
import jax.numpy as jnp
from jax import lax
from jax.experimental import pallas as pl
from jax.experimental.pallas import tpu as pltpu
from jax.experimental.pallas import tpu_sc as plsc

NUM_CORES = 2
NUM_SUBCORES = 16
NUM_WORKERS = NUM_CORES * NUM_SUBCORES
CHUNK = 128
G = 4


@functools.lru_cache(maxsize=None)
def _build(seq, vtiles, vocab, dim):
    dtiles = dim // 8
    n_blocks = seq * vtiles
    groups_per_w = n_blocks // (G * NUM_WORKERS)
    n_outer = groups_per_w // 2
    groups_per_row = vtiles // G
    mesh = plsc.VectorSubcoreMesh(
        core_axis_name="c", subcore_axis_name="s")

    @functools.partial(
        pl.kernel,
        out_type=jax.ShapeDtypeStruct(
            (seq, dtiles, vtiles, 8, CHUNK), jnp.float32),
        mesh=mesh,
        scratch_types=[
            pltpu.VMEM((G, CHUNK), jnp.int32),
            pltpu.VMEM((G, CHUNK), jnp.int32),
            pltpu.VMEM((G * CHUNK, dim), jnp.float32),
            pltpu.VMEM((G * CHUNK, dim), jnp.float32),
            pltpu.VMEM((dtiles, G, 8, CHUNK), jnp.float32),
            pltpu.VMEM((dtiles, G, 8, CHUNK), jnp.float32),
            pltpu.SemaphoreType.DMA,
            pltpu.SemaphoreType.DMA,
            pltpu.SemaphoreType.DMA,
            pltpu.SemaphoreType.DMA,
        ],
        compiler_params=pltpu.CompilerParams(
            use_tc_tiling_on_sc=False, needs_layout_passes=False,
            disable_bounds_checks=True, skip_device_barrier=True),
    )
    def emb(idx_hbm, table_hbm, out_hbm,
            idx0, idx1, rows0, rows1, trans0, trans1, g0, g1, o0, o1):
        wid = lax.axis_index("s") * NUM_CORES + lax.axis_index("c")
        gid0 = wid * groups_per_w
        idxv = (idx0, idx1)
        rows = (rows0, rows1)
        trans = (trans0, trans1)
        gsem = (g0, g1)
        osem = (o0, o1)
        def coords(gid):
            s = gid // groups_per_row
            vt0 = (gid % groups_per_row) * G
            return s, vt0

        def fire_group(gid, sl):
            s, vt0 = coords(gid)
            pltpu.sync_copy(
                idx_hbm.at[s // 8, pl.ds(vt0, G), s % 8], idxv[sl])
            for b in range(G):
                pltpu.async_copy(
                    table_hbm.at[idxv[sl].at[b]],
                    rows[sl].at[pl.ds(b * CHUNK, CHUNK)], gsem[sl])

        def drain_group(sl):
            for b in range(G):
                pltpu.make_async_copy(
                    table_hbm.at[idxv[sl].at[b]],
                    rows[sl].at[pl.ds(b * CHUNK, CHUNK)], gsem[sl]).wait()

        def transpose_group(sl):
            r = rows[sl]
            t = trans[sl]

            def bbody(b, carry):
                def tbody(tg, vv):
                    iota = lax.iota(jnp.int32, 16)
                    bv = jnp.zeros((16,), jnp.int32) + b
                    rowv = vv + b * CHUNK
                    vals = []
                    for d0 in range(dim):
                        colv = (d0 + iota) % dim
                        vals.append(plsc.load_gather(r, [rowv, colv]))
                    for d0 in range(dim):
                        colv = (d0 + iota) % dim
                        plsc.store_scatter(
                            t, [colv // 8, bv, colv % 8, vv], vals[d0])
                    return vv + 16

                lax.fori_loop(0, CHUNK // 16, tbody,
                              lax.iota(jnp.int32, 16))
                return carry

            lax.fori_loop(0, G, bbody, 0)

        def fire_store(gid, sl):
            s, vt0 = coords(gid)
            pltpu.async_copy(
                trans[sl], out_hbm.at[s, :, pl.ds(vt0, G)], osem[sl])

        def wait_store(gid, sl):
            s, vt0 = coords(gid)
            pltpu.make_async_copy(
                trans[sl], out_hbm.at[s, :, pl.ds(vt0, G)], osem[sl]).wait()

        fire_group(gid0, 0)

        def outer(ti, carry):
            for sl in range(2):
                g = ti * 2 + sl
                gid = gid0 + g

                @pl.when(g >= 2)
                def _wait_old_store():
                    wait_store(gid - 2, sl)

                @pl.when(g + 1 < groups_per_w)
                def _fire_next():
                    fire_group(gid + 1, 1 - sl)

                drain_group(sl)
                transpose_group(sl)
                fire_store(gid, sl)
            return carry

        lax.fori_loop(0, n_outer, outer, 0)
        wait_store(gid0 + groups_per_w - 2, 0)
        wait_store(gid0 + groups_per_w - 1, 1)

    return emb


def kernel(token_ids, weight):
    n, seq = token_ids.shape
    vocab, dim = weight.shape
    vtiles = n // CHUNK
    idx_t = (token_ids.T.reshape(seq // 8, 8, vtiles, CHUNK)
             .transpose(0, 2, 1, 3))
    out5 = _build(seq, vtiles, vocab, dim)(idx_t, weight)
    return out5.transpose(2, 4, 0, 1, 3).reshape(n, seq, dim)
